# baseline (speedup 1.0000x reference)
import functools

import jax
import jax.numpy as jnp
from jax import lax
from jax.experimental import pallas as pl
from jax.experimental.pallas import tpu as pltpu
from jax.experimental.pallas import tpu_sc as plsc

NC = 2
NS = 16
L = 16
NW = NC * NS
EDGE_BATCH = 128
NBUF = 3


@functools.lru_cache(maxsize=None)
def _build_sc_kernel(n_nodes, d_feat, e_pad):
  assert n_nodes % NS == 0 and d_feat % L == 0
  assert e_pad % (NW * EDGE_BATCH * NBUF) == 0
  epw = e_pad // NW
  n_batches = epw // EDGE_BATCH
  zt = 10
  rows_per_tile = n_nodes // zt
  assert rows_per_tile % 8 == 0
  n_chunks = d_feat // L
  n_groups = EDGE_BATCH // L

  mesh = plsc.VectorSubcoreMesh(core_axis_name="c", subcore_axis_name="s",
                                num_cores=NC)

  scratch = (
      [pltpu.VMEM_SHARED((n_nodes, d_feat), jnp.float32)]
      + [pltpu.VMEM((EDGE_BATCH,), jnp.int32)] * NBUF
      + [pltpu.VMEM((EDGE_BATCH,), jnp.int32)] * NBUF
      + [pltpu.VMEM((EDGE_BATCH,), jnp.float32)] * NBUF
      + [pltpu.VMEM((EDGE_BATCH,), jnp.int32)] * NBUF
      + [pltpu.VMEM((EDGE_BATCH, d_feat), jnp.float32)] * NBUF
      + [pltpu.SemaphoreType.DMA] * (3 * NBUF)
  )

  @functools.partial(
      pl.kernel,
      out_type=jax.ShapeDtypeStruct((NC, n_nodes, d_feat), jnp.float32),
      mesh=mesh,
      scratch_types=scratch,
  )
  def sc_kernel(x_hbm, src_hbm, dst_hbm, w_hbm, out_hbm, acc, *sc):
    sidx = sc[0:NBUF]
    didx = sc[NBUF:2 * NBUF]
    wv = sc[2 * NBUF:3 * NBUF]
    dsc = sc[3 * NBUF:4 * NBUF]
    rows = sc[4 * NBUF:5 * NBUF]
    semi = sc[5 * NBUF:6 * NBUF]
    semg = sc[6 * NBUF:7 * NBUF]
    sems = sc[7 * NBUF:8 * NBUF]

    cid = lax.axis_index("c")
    sid = lax.axis_index("s")
    wid = cid * NS + sid
    ebase = wid * epw

    @pl.when(sid < zt)
    def _():
      def zfill(j, _):
        for c in range(n_chunks):
          rows[0][j, pl.ds(c * L, L)] = jnp.zeros((L,), jnp.float32)
        return 0
      lax.fori_loop(0, EDGE_BATCH, zfill, 0)

      full, rem = divmod(rows_per_tile, EDGE_BATCH)
      for j in range(full):
        pltpu.sync_copy(
            rows[0],
            acc.at[pl.ds(sid * rows_per_tile + j * EDGE_BATCH, EDGE_BATCH)])
      if rem:
        pltpu.sync_copy(
            rows[0].at[pl.ds(0, rem)],
            acc.at[pl.ds(sid * rows_per_tile + full * EDGE_BATCH, rem)])
    plsc.subcore_barrier()

    def start_idx(b, p):
      off = ebase + b * EDGE_BATCH
      pltpu.async_copy(src_hbm.at[pl.ds(off, EDGE_BATCH)], sidx[p], semi[p])
      pltpu.async_copy(dst_hbm.at[pl.ds(off, EDGE_BATCH)], didx[p], semi[p])
      pltpu.async_copy(w_hbm.at[pl.ds(off, EDGE_BATCH)], wv[p], semi[p])

    def wait_idx(p):
      pltpu.make_async_copy(src_hbm.at[pl.ds(0, EDGE_BATCH)], sidx[p],
                            semi[p]).wait()
      pltpu.make_async_copy(dst_hbm.at[pl.ds(0, EDGE_BATCH)], didx[p],
                            semi[p]).wait()
      pltpu.make_async_copy(w_hbm.at[pl.ds(0, EDGE_BATCH)], wv[p],
                            semi[p]).wait()

    def start_gather(p):
      pltpu.async_copy(x_hbm.at[sidx[p]], rows[p], semg[p])

    def wait_gather(p):
      pltpu.make_async_copy(x_hbm.at[sidx[p]], rows[p], semg[p]).wait()

    def start_scatter(p):
      for c in range(n_groups):
        dsc[p][pl.ds(c * L, L)] = didx[p][pl.ds(c * L, L)]
      pltpu.async_copy(rows[p], acc.at[dsc[p]], sems[p], add=True)

    def wait_scatter(p):
      pltpu.make_async_copy(rows[p], acc.at[dsc[p]], sems[p]).wait()

    def scale(p):
      def scale_group(g, _):
        wvec = wv[p][pl.ds(g * L, L)]
        for jj in range(L):
          s = jnp.full((L,), wvec[jj], jnp.float32)
          j = g * L + jj
          for c in range(n_chunks):
            rows[p][j, pl.ds(c * L, L)] = rows[p][j, pl.ds(c * L, L)] * s
        return 0
      lax.fori_loop(0, n_groups, scale_group, 0)

    for p in range(NBUF):
      start_idx(p, p)
    wait_idx(0)
    start_gather(0)
    wait_idx(1)
    start_gather(1)

    def outer(k, _):
      b0 = k * NBUF
      for p in range(NBUF):
        b = b0 + p
        q = (p + 2) % NBUF

        @pl.when(b + 2 < n_batches)
        def _():
          wait_idx(q)

          @pl.when(b >= 1)
          def _():
            wait_scatter(q)
          start_gather(q)

        wait_gather(p)
        scale(p)
        start_scatter(p)

        @pl.when(b + NBUF < n_batches)
        def _():
          start_idx(b + NBUF, p)
      return 0
    lax.fori_loop(0, n_batches // NBUF, outer, 0)

    for p in range(NBUF):
      wait_scatter(p)
    plsc.subcore_barrier()

    @pl.when(sid < zt)
    def _():
      pltpu.sync_copy(
          acc.at[pl.ds(sid * rows_per_tile, rows_per_tile)],
          out_hbm.at[cid, pl.ds(sid * rows_per_tile, rows_per_tile)])

  return sc_kernel


def _combine_body(p_ref, o_ref):
  o_ref[...] = p_ref[0] + p_ref[1]


@functools.lru_cache(maxsize=None)
def _build_combine(n_nodes, d_feat):
  grid = 10 if n_nodes % 80 == 0 else 1
  blk = n_nodes // grid
  return pl.pallas_call(
      _combine_body,
      grid=(grid,),
      in_specs=[pl.BlockSpec((NC, blk, d_feat), lambda i: (0, i, 0))],
      out_specs=pl.BlockSpec((blk, d_feat), lambda i: (i, 0)),
      out_shape=jax.ShapeDtypeStruct((n_nodes, d_feat), jnp.float32),
  )


def kernel(x, edge_index, edge_weight):
  n_nodes, d_feat = x.shape
  n_edges = edge_index.shape[1]
  src = edge_index[0].astype(jnp.int32)
  dst = edge_index[1].astype(jnp.int32)
  w = edge_weight.astype(jnp.float32)

  chunk = NW * EDGE_BATCH * NBUF
  e_pad = ((n_edges + chunk - 1) // chunk) * chunk
  if e_pad != n_edges:
    pad = e_pad - n_edges
    src = jnp.concatenate([src, jnp.zeros((pad,), jnp.int32)])
    dst = jnp.concatenate([dst, jnp.zeros((pad,), jnp.int32)])
    w = jnp.concatenate([w, jnp.zeros((pad,), jnp.float32)])

  partial = _build_sc_kernel(n_nodes, d_feat, e_pad)(x, src, dst, w)
  return _build_combine(n_nodes, d_feat)(partial)


# speedup vs baseline: 1.3439x; 1.3439x over previous
import functools

import jax
import jax.numpy as jnp
from jax import lax
from jax.experimental import pallas as pl
from jax.experimental.pallas import tpu as pltpu
from jax.experimental.pallas import tpu_sc as plsc

NC = 2
NS = 16
L = 16
NW = NC * NS
EDGE_BATCH = 128
NBUF = 2


@functools.lru_cache(maxsize=None)
def _build_sc_kernel(n_nodes, d_feat, e_pad):
  assert n_nodes % NS == 0 and d_feat % L == 0
  assert e_pad % (NW * EDGE_BATCH * NBUF) == 0
  epw = e_pad // NW
  n_batches = epw // EDGE_BATCH
  zt = 10
  rows_per_tile = n_nodes // zt
  assert rows_per_tile % 8 == 0
  n_chunks = d_feat // L
  n_groups = EDGE_BATCH // L

  mesh = plsc.VectorSubcoreMesh(core_axis_name="c", subcore_axis_name="s",
                                num_cores=NC)

  scratch = (
      [pltpu.VMEM_SHARED((n_nodes, d_feat), jnp.float32)]
      + [pltpu.VMEM((EDGE_BATCH,), jnp.int32)] * NBUF
      + [pltpu.VMEM((EDGE_BATCH,), jnp.int32)] * NBUF
      + [pltpu.VMEM((EDGE_BATCH,), jnp.float32)] * NBUF
      + [pltpu.VMEM((EDGE_BATCH, d_feat), jnp.float32)] * NBUF
      + [pltpu.SemaphoreType.DMA] * (2 * NBUF)
  )

  @functools.partial(
      pl.kernel,
      out_type=jax.ShapeDtypeStruct((NC, n_nodes, d_feat), jnp.float32),
      mesh=mesh,
      scratch_types=scratch,
  )
  def sc_kernel(x_hbm, src_hbm, dst_hbm, w_hbm, out_hbm, acc, *sc):
    sidx = sc[0:NBUF]
    didx = sc[NBUF:2 * NBUF]
    wv = sc[2 * NBUF:3 * NBUF]
    rows = sc[3 * NBUF:4 * NBUF]
    semi = sc[4 * NBUF:5 * NBUF]
    semg = sc[5 * NBUF:6 * NBUF]

    cid = lax.axis_index("c")
    sid = lax.axis_index("s")
    wid = cid * NS + sid
    ebase = wid * epw

    @pl.when(sid < zt)
    def _():
      def zfill(j, _):
        for c in range(n_chunks):
          rows[0][j, pl.ds(c * L, L)] = jnp.zeros((L,), jnp.float32)
        return 0
      lax.fori_loop(0, EDGE_BATCH, zfill, 0)

      full, rem = divmod(rows_per_tile, EDGE_BATCH)
      for j in range(full):
        pltpu.sync_copy(
            rows[0],
            acc.at[pl.ds(sid * rows_per_tile + j * EDGE_BATCH, EDGE_BATCH)])
      if rem:
        pltpu.sync_copy(
            rows[0].at[pl.ds(0, rem)],
            acc.at[pl.ds(sid * rows_per_tile + full * EDGE_BATCH, rem)])
    plsc.subcore_barrier()

    def start_idx(b, p):
      off = ebase + b * EDGE_BATCH
      pltpu.async_copy(src_hbm.at[pl.ds(off, EDGE_BATCH)], sidx[p], semi[p])
      pltpu.async_copy(dst_hbm.at[pl.ds(off, EDGE_BATCH)], didx[p], semi[p])
      pltpu.async_copy(w_hbm.at[pl.ds(off, EDGE_BATCH)], wv[p], semi[p])

    def wait_idx(p):
      pltpu.make_async_copy(src_hbm.at[pl.ds(0, EDGE_BATCH)], sidx[p],
                            semi[p]).wait()
      pltpu.make_async_copy(dst_hbm.at[pl.ds(0, EDGE_BATCH)], didx[p],
                            semi[p]).wait()
      pltpu.make_async_copy(w_hbm.at[pl.ds(0, EDGE_BATCH)], wv[p],
                            semi[p]).wait()

    def start_gather(p):
      pltpu.async_copy(x_hbm.at[sidx[p]], rows[p], semg[p])

    def wait_gather(p):
      pltpu.make_async_copy(x_hbm.at[sidx[p]], rows[p], semg[p]).wait()

    def scale(p):
      def scale_group(g, _):
        wvec = wv[p][pl.ds(g * L, L)]
        for jj in range(L):
          s = jnp.full((L,), wvec[jj], jnp.float32)
          j = g * L + jj
          for c in range(n_chunks):
            rows[p][j, pl.ds(c * L, L)] = rows[p][j, pl.ds(c * L, L)] * s
        return 0
      lax.fori_loop(0, n_groups, scale_group, 0)

    start_idx(0, 0)
    start_idx(1, 1)
    wait_idx(0)
    start_gather(0)

    def outer(k, _):
      b0 = k * NBUF
      for p in range(NBUF):
        b = b0 + p
        q = (p + 1) % NBUF

        wait_gather(p)

        @pl.when(b + 1 < n_batches)
        def _():
          wait_idx(q)
          start_gather(q)

        scale(p)
        pltpu.sync_copy(rows[p], acc.at[didx[p]], add=True)

        @pl.when(b + NBUF < n_batches)
        def _():
          start_idx(b + NBUF, p)
      return 0
    lax.fori_loop(0, n_batches // NBUF, outer, 0)
    plsc.subcore_barrier()

    @pl.when(sid < zt)
    def _():
      pltpu.sync_copy(
          acc.at[pl.ds(sid * rows_per_tile, rows_per_tile)],
          out_hbm.at[cid, pl.ds(sid * rows_per_tile, rows_per_tile)])

  return sc_kernel


def _combine_body(p_ref, o_ref):
  o_ref[...] = p_ref[0] + p_ref[1]


@functools.lru_cache(maxsize=None)
def _build_combine(n_nodes, d_feat):
  grid = 10 if n_nodes % 80 == 0 else 1
  blk = n_nodes // grid
  return pl.pallas_call(
      _combine_body,
      grid=(grid,),
      in_specs=[pl.BlockSpec((NC, blk, d_feat), lambda i: (0, i, 0))],
      out_specs=pl.BlockSpec((blk, d_feat), lambda i: (i, 0)),
      out_shape=jax.ShapeDtypeStruct((n_nodes, d_feat), jnp.float32),
  )


def kernel(x, edge_index, edge_weight):
  n_nodes, d_feat = x.shape
  n_edges = edge_index.shape[1]
  src = edge_index[0].astype(jnp.int32)
  dst = edge_index[1].astype(jnp.int32)
  w = edge_weight.astype(jnp.float32)

  chunk = NW * EDGE_BATCH * NBUF
  e_pad = ((n_edges + chunk - 1) // chunk) * chunk
  if e_pad != n_edges:
    pad = e_pad - n_edges
    src = jnp.concatenate([src, jnp.zeros((pad,), jnp.int32)])
    dst = jnp.concatenate([dst, jnp.zeros((pad,), jnp.int32)])
    w = jnp.concatenate([w, jnp.zeros((pad,), jnp.float32)])

  partial = _build_sc_kernel(n_nodes, d_feat, e_pad)(x, src, dst, w)
  return _build_combine(n_nodes, d_feat)(partial)


# speedup vs baseline: 1.5991x; 1.1899x over previous
import functools

import jax
import jax.numpy as jnp
from jax import lax
from jax.experimental import pallas as pl
from jax.experimental.pallas import tpu as pltpu
from jax.experimental.pallas import tpu_sc as plsc

NC = 2
NS = 16
L = 16
NW = NC * NS
EDGE_BATCH = 128


@functools.lru_cache(maxsize=None)
def _build_sc_kernel(n_nodes, d_feat, e_pad):
  assert n_nodes % NS == 0 and d_feat % L == 0
  assert e_pad % (NW * EDGE_BATCH) == 0
  epw = e_pad // NW
  n_batches = epw // EDGE_BATCH
  nb_total = e_pad // EDGE_BATCH
  zt = 10
  rows_per_tile = n_nodes // zt
  assert rows_per_tile % 8 == 0
  n_chunks = d_feat // L
  n_groups = EDGE_BATCH // L

  mesh = plsc.VectorSubcoreMesh(core_axis_name="c", subcore_axis_name="s",
                                num_cores=NC)

  @functools.partial(
      pl.kernel,
      out_type=jax.ShapeDtypeStruct((NC, n_nodes, d_feat), jnp.float32),
      mesh=mesh,
      scratch_types=[
          pltpu.VMEM_SHARED((n_nodes, d_feat), jnp.float32),
          pltpu.VMEM((3, EDGE_BATCH), jnp.int32),
          pltpu.VMEM((EDGE_BATCH, d_feat), jnp.float32),
          pltpu.SemaphoreType.DMA,
      ],
  )
  def sc_kernel(x_hbm, pack_hbm, out_hbm, acc, pkv, rows_v, sem):
    cid = lax.axis_index("c")
    sid = lax.axis_index("s")
    wid = cid * NS + sid

    @pl.when(sid < zt)
    def _():
      def zfill(j, _):
        for c in range(n_chunks):
          rows_v[j, pl.ds(c * L, L)] = jnp.zeros((L,), jnp.float32)
        return 0
      lax.fori_loop(0, EDGE_BATCH, zfill, 0)

      full, rem = divmod(rows_per_tile, EDGE_BATCH)
      for j in range(full):
        pltpu.sync_copy(
            rows_v,
            acc.at[pl.ds(sid * rows_per_tile + j * EDGE_BATCH, EDGE_BATCH)])
      if rem:
        pltpu.sync_copy(
            rows_v.at[pl.ds(0, rem)],
            acc.at[pl.ds(sid * rows_per_tile + full * EDGE_BATCH, rem)])
    plsc.subcore_barrier()

    bbase = wid * n_batches

    def edge_batch(b, _):
      pltpu.sync_copy(pack_hbm.at[bbase + b], pkv)
      pltpu.async_copy(x_hbm.at[pkv.at[0]], rows_v, sem).wait()

      def scale_group(g, _):
        wvec = pkv[2, pl.ds(g * L, L)]
        for jj in range(L):
          s = jnp.full((L,), lax.bitcast_convert_type(wvec[jj], jnp.float32),
                       jnp.float32)
          j = g * L + jj
          for c in range(n_chunks):
            rows_v[j, pl.ds(c * L, L)] = rows_v[j, pl.ds(c * L, L)] * s
        return 0
      lax.fori_loop(0, n_groups, scale_group, 0)

      pltpu.sync_copy(rows_v, acc.at[pkv.at[1]], add=True)
      return 0
    lax.fori_loop(0, n_batches, edge_batch, 0)
    plsc.subcore_barrier()

    @pl.when(sid < zt)
    def _():
      pltpu.sync_copy(
          acc.at[pl.ds(sid * rows_per_tile, rows_per_tile)],
          out_hbm.at[cid, pl.ds(sid * rows_per_tile, rows_per_tile)])

  return sc_kernel


def _combine_body(p_ref, o_ref):
  o_ref[...] = p_ref[0] + p_ref[1]


@functools.lru_cache(maxsize=None)
def _build_combine(n_nodes, d_feat):
  grid = 10 if n_nodes % 80 == 0 else 1
  blk = n_nodes // grid
  return pl.pallas_call(
      _combine_body,
      grid=(grid,),
      in_specs=[pl.BlockSpec((NC, blk, d_feat), lambda i: (0, i, 0))],
      out_specs=pl.BlockSpec((blk, d_feat), lambda i: (i, 0)),
      out_shape=jax.ShapeDtypeStruct((n_nodes, d_feat), jnp.float32),
  )


def kernel(x, edge_index, edge_weight):
  n_nodes, d_feat = x.shape
  n_edges = edge_index.shape[1]
  src = edge_index[0].astype(jnp.int32)
  dst = edge_index[1].astype(jnp.int32)
  w = edge_weight.astype(jnp.float32)

  chunk = NW * EDGE_BATCH
  e_pad = ((n_edges + chunk - 1) // chunk) * chunk
  if e_pad != n_edges:
    pad = e_pad - n_edges
    src = jnp.concatenate([src, jnp.zeros((pad,), jnp.int32)])
    dst = jnp.concatenate([dst, jnp.zeros((pad,), jnp.int32)])
    w = jnp.concatenate([w, jnp.zeros((pad,), jnp.float32)])

  nb_total = e_pad // EDGE_BATCH
  pack = jnp.stack(
      [src.reshape(nb_total, EDGE_BATCH),
       dst.reshape(nb_total, EDGE_BATCH),
       lax.bitcast_convert_type(w, jnp.int32).reshape(nb_total, EDGE_BATCH)],
      axis=1)

  partial = _build_sc_kernel(n_nodes, d_feat, e_pad)(x, pack)
  return _build_combine(n_nodes, d_feat)(partial)
